# baseline (speedup 1.0000x reference)
import functools

import jax
import jax.numpy as jnp
from jax import lax
from jax.experimental import pallas as pl
from jax.experimental.pallas import tpu as pltpu
from jax.experimental.pallas import tpu_sc as plsc

MAX_LEN = 512
D_MODEL = 64
BATCH = 16384

_NUM_CORES = 1
_NUM_SUBCORES = 16
_NUM_WORKERS = _NUM_CORES * _NUM_SUBCORES
_B_PER_W = BATCH // _NUM_WORKERS

_CHUNKS = 2
_C = _B_PER_W // _CHUNKS

_mesh = plsc.VectorSubcoreMesh(
    core_axis_name="c", subcore_axis_name="s",
    num_cores=_NUM_CORES, num_subcores=_NUM_SUBCORES,
)


@functools.partial(
    pl.kernel,
    out_type=jax.ShapeDtypeStruct((BATCH, D_MODEL), jnp.float32),
    mesh=_mesh,
    compiler_params=pltpu.CompilerParams(use_tc_tiling_on_sc=False),
    scratch_types=[
        pltpu.VMEM((_B_PER_W,), jnp.int32),
        pltpu.VMEM((_CHUNKS, _C, D_MODEL), jnp.float32),
        [pltpu.SemaphoreType.DMA] * _CHUNKS,
        [pltpu.SemaphoreType.DMA] * _CHUNKS,
    ],
)
def _sc_gather(table_hbm, idx_hbm, out_hbm, idx_v, rows_v, gsems, wsems):
    wid = lax.axis_index("s") * _NUM_CORES + lax.axis_index("c")
    base = wid * _B_PER_W
    pltpu.sync_copy(idx_hbm.at[pl.ds(base, _B_PER_W)], idx_v)
    gathers = [
        pltpu.async_copy(
            table_hbm.at[idx_v.at[pl.ds(c * _C, _C)]], rows_v.at[c], gsems[c]
        )
        for c in range(_CHUNKS)
    ]
    writes = []
    for c in range(_CHUNKS):
        gathers[c].wait()
        writes.append(
            pltpu.async_copy(
                rows_v.at[c], out_hbm.at[pl.ds(base + c * _C, _C)], wsems[c]
            )
        )
    for w in writes:
        w.wait()


def kernel(positions, table):
    return _sc_gather(table, positions.astype(jnp.int32))


# speedup vs baseline: 1.0001x; 1.0001x over previous
import functools

import jax
import jax.numpy as jnp
from jax import lax
from jax.experimental import pallas as pl
from jax.experimental.pallas import tpu as pltpu
from jax.experimental.pallas import tpu_sc as plsc

MAX_LEN = 512
D_MODEL = 64
BATCH = 16384

_NUM_CORES = 1
_NUM_SUBCORES = 16
_NUM_WORKERS = _NUM_CORES * _NUM_SUBCORES
_B_PER_W = BATCH // _NUM_WORKERS

_CHUNKS = 2
_C = _B_PER_W // _CHUNKS

_mesh = plsc.VectorSubcoreMesh(
    core_axis_name="c", subcore_axis_name="s",
    num_cores=_NUM_CORES, num_subcores=_NUM_SUBCORES,
)


@functools.partial(
    pl.kernel,
    out_type=jax.ShapeDtypeStruct((BATCH, D_MODEL), jnp.float32),
    mesh=_mesh,
    compiler_params=pltpu.CompilerParams(use_tc_tiling_on_sc=False),
    scratch_types=[
        pltpu.VMEM((_B_PER_W,), jnp.int32),
        pltpu.VMEM((_CHUNKS, _C, D_MODEL), jnp.float32),
        [pltpu.SemaphoreType.DMA] * _CHUNKS,
        [pltpu.SemaphoreType.DMA] * _CHUNKS,
        [pltpu.SemaphoreType.DMA] * _CHUNKS,
    ],
)
def _sc_gather(table_hbm, idx_hbm, out_hbm, idx_v, rows_v, gsems, wsems, isems):
    wid = lax.axis_index("s") * _NUM_CORES + lax.axis_index("c")
    base = wid * _B_PER_W
    idx_copies = [
        pltpu.async_copy(
            idx_hbm.at[pl.ds(base + c * _C, _C)],
            idx_v.at[pl.ds(c * _C, _C)],
            isems[c],
        )
        for c in range(_CHUNKS)
    ]
    gathers = []
    for c in range(_CHUNKS):
        idx_copies[c].wait()
        gathers.append(
            pltpu.async_copy(
                table_hbm.at[idx_v.at[pl.ds(c * _C, _C)]], rows_v.at[c], gsems[c]
            )
        )
    writes = []
    for c in range(_CHUNKS):
        gathers[c].wait()
        writes.append(
            pltpu.async_copy(
                rows_v.at[c], out_hbm.at[pl.ds(base + c * _C, _C)], wsems[c]
            )
        )
    for w in writes:
        w.wait()


def kernel(positions, table):
    return _sc_gather(table, positions.astype(jnp.int32))
